# baseline (speedup 1.0000x reference)
import functools

import jax
import jax.numpy as jnp
import numpy as np
from jax import lax
from jax.experimental import pallas as pl
from jax.experimental.pallas import tpu as pltpu
from jax.experimental.pallas import tpu_sc as plsc

NUM_LAYERS = 4
D_OUT = [16, 64, 128, 256]
D_FEATURE = 8
NUM_CLASSES = 13
K_N = 16
B = 2
N0 = 65536
SUB = 4
NS = [N0 // (SUB ** i) for i in range(NUM_LAYERS + 1)]
BN_SCALE = 1.0 / np.sqrt(1.0 + 1e-6)
NW = 32


def _leaky(x):
    return jnp.where(x >= 0, x, 0.2 * x)


@functools.lru_cache(maxsize=None)
def _make_sc_gather(R, d, M, seg, n_tab):
    chunk = M // NW
    assert M % NW == 0 and seg % chunk == 0
    d_eff = max(8, d)
    S = min(chunk, 8192)
    while S * (d_eff + 1) > 98304:
        S //= 2
    n_sub = chunk // S
    mesh = plsc.VectorSubcoreMesh(core_axis_name="c", subcore_axis_name="s")

    @functools.partial(
        pl.kernel, mesh=mesh,
        out_type=jax.ShapeDtypeStruct((M, d), jnp.float32),
        scratch_types=[pltpu.VMEM((S,), jnp.int32),
                       pltpu.VMEM((S, d), jnp.float32),
                       pltpu.SemaphoreType.DMA],
        compiler_params=pltpu.CompilerParams(use_tc_tiling_on_sc=False))
    def k(table_hbm, idx_hbm, out_hbm, idx_v, rows_v, sem):
        c = lax.axis_index("c")
        s = lax.axis_index("s")
        wid = s * 2 + c
        base = wid * chunk
        boff = ((base // seg) % B) * n_tab

        def body(j, _):
            off = base + j * S
            pltpu.sync_copy(idx_hbm.at[pl.ds(off, S)], idx_v)

            def addb(t, _):
                sl = pl.ds(t * 16, 16)
                idx_v[sl] = idx_v[sl] + boff
                return 0

            lax.fori_loop(0, S // 16, addb, 0, unroll=8)
            pltpu.async_copy(table_hbm.at[idx_v], rows_v, sem).wait()
            pltpu.sync_copy(rows_v, out_hbm.at[pl.ds(off, S)])
            return 0

        if n_sub == 1:
            body(0, 0)
        else:
            lax.fori_loop(0, n_sub, body, 0)

    return k


def _sc_gather(table, idx, seg, n_tab):
    R, d = table.shape
    (M,) = idx.shape
    return _make_sc_gather(R, d, M, seg, n_tab)(table, idx)


def _bn(y, g, b):
    return g * y * BN_SCALE + b


def _mm(x, w):
    return jax.lax.dot_general(x, w, (((x.ndim - 1,), (0,)), ((), ())),
                               preferred_element_type=jnp.float32)


def _mm3(x, w):
    K, P, a = x.shape
    return _mm(x.reshape(K * P, a), w).reshape(K, P, w.shape[1])


def _w_full(shp):
    return pl.BlockSpec(shp, lambda *_: tuple(0 for _ in shp))


def _row_spec(P, d):
    return pl.BlockSpec((P, d), lambda i: (i, 0))


def _kp_spec(P, d):
    return pl.BlockSpec((K_N, P, d), lambda i: (0, i, 0))


def _halvmax(x, d_target):
    while x.shape[1] > d_target:
        w = x.shape[1] // 2
        x = jnp.maximum(x[:, :w], x[:, w:])
    return x


def _tc1_l0(feats, xyz, p, P=2048):
    BN = feats.shape[0]

    def body(x_ref, xyz_ref, w0, b0, g0, be0, w1, b1, g1, be1, f0_ref, xf_ref):
        y = _leaky(_bn(_mm(x_ref[...], w0[...]) + b0[...], g0[...], be0[...]))
        f0_ref[...] = y
        fpc = _leaky(_bn(_mm(y, w1[...]) + b1[...], g1[...], be1[...]))
        xf_ref[...] = jnp.concatenate([xyz_ref[...], fpc], axis=-1)

    return pl.pallas_call(
        body, grid=(BN // P,),
        in_specs=[_row_spec(P, 6), _row_spec(P, 8),
                  _w_full((6, 8)), _w_full((1, 8)), _w_full((1, 8)), _w_full((1, 8)),
                  _w_full((8, 8)), _w_full((1, 8)), _w_full((1, 8)), _w_full((1, 8))],
        out_specs=[_row_spec(P, 8), _row_spec(P, 16)],
        out_shape=[jax.ShapeDtypeStruct((BN, 8), jnp.float32),
                   jax.ShapeDtypeStruct((BN, 16), jnp.float32)],
    )(feats, xyz, p['fc0_W'], p['fc0_b'], p['bn0_gamma'], p['bn0_beta'],
      p['enc0_mlp1_W'], p['enc0_mlp1_b'], p['enc0_mlp1_gamma'], p['enc0_mlp1_beta'])


def _tc1(pg, xyz, p, nm, d_in, d2, P):
    BN = pg.shape[0]

    def body(pg_ref, xyz_ref, w1, b1, g1, be1, feat_ref, xf_ref):
        feat = _halvmax(pg_ref[...], d_in)
        feat_ref[...] = feat
        fpc = _leaky(_bn(_mm(feat, w1[...]) + b1[...], g1[...], be1[...]))
        xf_ref[...] = jnp.concatenate([xyz_ref[...], fpc], axis=-1)

    return pl.pallas_call(
        body, grid=(BN // P,),
        in_specs=[_row_spec(P, K_N * d_in), _row_spec(P, 8),
                  _w_full((d_in, d2)), _w_full((1, d2)), _w_full((1, d2)), _w_full((1, d2))],
        out_specs=[_row_spec(P, d_in), _row_spec(P, 8 + d2)],
        out_shape=[jax.ShapeDtypeStruct((BN, d_in), jnp.float32),
                   jax.ShapeDtypeStruct((BN, 8 + d2), jnp.float32)],
    )(pg, xyz, p[nm + 'mlp1_W'], p[nm + 'mlp1_b'], p[nm + 'mlp1_gamma'], p[nm + 'mlp1_beta'])


def _packed_consts(params, i):
    nm = 'enc%d_' % i
    d2 = D_OUT[i] // 2
    dout = D_OUT[i]
    K = K_N
    I_K = np.eye(K, dtype=np.float32)
    ones_K = np.ones((K, 1), dtype=np.float32)
    eye8 = np.eye(8, dtype=np.float32)
    Id2 = np.eye(d2, dtype=np.float32)
    Ido = np.eye(dout, dtype=np.float32)
    kron = jnp.kron
    tile = lambda a: jnp.tile(a.reshape(1, -1), (1, K))

    w10 = params[nm + 'lfa_mlp1_W']
    pad5 = jnp.zeros((5, d2), jnp.float32)
    wdis = w10[0:1]
    wrel8 = jnp.concatenate([w10[1:4], pad5], axis=0)
    wxyz8 = jnp.concatenate([w10[4:7], pad5], axis=0)
    wngx8 = jnp.concatenate([w10[7:10], pad5], axis=0)

    q = 8 + d2
    zx = jnp.zeros((d2, d2), jnp.float32)
    c = {}
    c['T8'] = jnp.asarray(np.tile(np.hstack([eye8, np.zeros((8, d2), np.float32)]),
                                  (1, K)))
    c['G8'] = jnp.asarray(np.kron(I_K, np.vstack([np.ones((8, 1), np.float32),
                                                  np.zeros((d2, 1), np.float32)])))
    c['Wxc'] = jnp.tile(wrel8 + wxyz8, (1, K))
    c['BDgx'] = kron(jnp.asarray(I_K),
                     jnp.concatenate([wngx8 - wrel8, zx], axis=0))
    c['SELf'] = jnp.asarray(np.kron(I_K, np.vstack(
        [np.zeros((8, d2), np.float32), Id2])))
    c['KW'] = kron(jnp.asarray(I_K), wdis)
    c['b1t'] = tile(params[nm + 'lfa_mlp1_b'])
    c['g1t'] = tile(params[nm + 'lfa_mlp1_gamma'])
    c['be1t'] = tile(params[nm + 'lfa_mlp1_beta'])

    c['Gdo'] = jnp.asarray(np.kron(ones_K, Ido))
    c['Tdo'] = jnp.asarray(np.tile(Ido, (1, K)))
    sel_lo = np.kron(I_K, np.vstack([Id2, np.zeros((d2, d2), np.float32)]))
    sel_hi = np.kron(I_K, np.vstack([np.zeros((d2, d2), np.float32), Id2]))
    c['SEL_lo'] = jnp.asarray(sel_lo)
    c['SEL_hi'] = jnp.asarray(sel_hi)

    for att, od in (('att1', d2), ('att2', dout)):
        W = params[nm + att + '_fc_W']
        c[att + '_BDfc_g'] = kron(jnp.asarray(I_K), W[:d2])
        c[att + '_BDfc_x'] = kron(jnp.asarray(I_K), W[d2:])
        c[att + '_bfct'] = tile(params[nm + att + '_fc_b'])
        Wm = params[nm + att + '_mlp_W']
        c[att + '_GWam_lo'] = kron(jnp.asarray(ones_K), Wm[:d2])
        c[att + '_GWam_hi'] = kron(jnp.asarray(ones_K), Wm[d2:])
        r2 = lambda a: a.reshape(1, -1)
        c[att + '_bam'] = r2(params[nm + att + '_mlp_b'])
        c[att + '_gam'] = r2(params[nm + att + '_mlp_gamma'])
        c[att + '_beam'] = r2(params[nm + att + '_mlp_beta'])

    c['BDl2'] = kron(jnp.asarray(I_K), params[nm + 'lfa_mlp2_W'])
    c['b2t'] = tile(params[nm + 'lfa_mlp2_b'])
    c['g2t'] = tile(params[nm + 'lfa_mlp2_gamma'])
    c['be2t'] = tile(params[nm + 'lfa_mlp2_beta'])

    r2 = lambda a: a.reshape(1, -1)
    for s in ('mlp2', 'shortcut'):
        c[s + '_W'] = params[nm + s + '_W']
        c[s + '_b'] = r2(params[nm + s + '_b'])
        c[s + '_g'] = r2(params[nm + s + '_gamma'])
        c[s + '_be'] = r2(params[nm + s + '_beta'])
    return c


def _pk_lfa1(xyz, gx, c):
    xyzt = _mm(xyz, c['T8'])
    rel = xyzt - gx
    dis = jnp.sqrt(_mm(rel * rel, c['G8']) + 1e-12)
    t = _mm(xyz, c['Wxc']) + _mm(gx, c['BDgx']) + _mm(dis, c['KW']) + c['b1t']
    return _leaky(c['g1t'] * t * BN_SCALE + c['be1t'])


def _pk_att(fa, fb, c, att):
    A = _mm(fa, c[att + '_BDfc_g']) + _mm(fb, c[att + '_BDfc_x']) + c[att + '_bfct']
    A = A - jnp.max(A, axis=-1, keepdims=True)
    E = jnp.exp(A)
    rden = 1.0 / _mm(E, c['Gdo'])
    S = E * _mm(rden, c['Tdo'])
    agg = (_mm(fa * _mm(S, c['SEL_lo']), c[att + '_GWam_lo'])
           + _mm(fb * _mm(S, c['SEL_hi']), c[att + '_GWam_hi']))
    return _leaky(_bn(agg + c[att + '_bam'], c[att + '_gam'], c[att + '_beam']))


_PK_KEYS2 = ['T8', 'G8', 'Wxc', 'BDgx', 'SELf', 'KW', 'b1t', 'g1t', 'be1t',
             'Gdo', 'Tdo', 'SEL_lo', 'SEL_hi',
             'att1_BDfc_g', 'att1_BDfc_x', 'att1_bfct',
             'att1_GWam_lo', 'att1_GWam_hi', 'att1_bam', 'att1_gam', 'att1_beam']

_PK_KEYS3 = ['T8', 'G8', 'Wxc', 'BDgx', 'KW', 'b1t', 'g1t', 'be1t',
             'BDl2', 'b2t', 'g2t', 'be2t',
             'Gdo', 'Tdo', 'SEL_lo', 'SEL_hi',
             'att2_BDfc_g', 'att2_BDfc_x', 'att2_bfct',
             'att2_GWam_lo', 'att2_GWam_hi', 'att2_bam', 'att2_gam', 'att2_beam',
             'mlp2_W', 'mlp2_b', 'mlp2_g', 'mlp2_be',
             'shortcut_W', 'shortcut_b', 'shortcut_g', 'shortcut_be']


def _tc2_packed(xyzp, gxg, c, P):
    BN = xyzp.shape[0]
    d2 = c['KW'].shape[1] // K_N
    consts = [c[k] for k in _PK_KEYS2]

    def body(xyz_ref, gxg_ref, *refs):
        cd = {k: r[...] for k, r in zip(_PK_KEYS2, refs[:-1])}
        out_ref = refs[-1]
        gxg = gxg_ref[...]
        fx = _pk_lfa1(xyz_ref[...], gxg, cd)
        g1p = _mm(gxg, cd['SELf'])
        out_ref[...] = _pk_att(g1p, fx, cd, 'att1')

    return pl.pallas_call(
        body, grid=(BN // P,),
        in_specs=[_row_spec(P, 8), _row_spec(P, K_N * (8 + d2))]
                 + [_w_full(x.shape) for x in consts],
        out_specs=_row_spec(P, d2),
        out_shape=jax.ShapeDtypeStruct((BN, d2), jnp.float32),
    )(xyzp, gxg, *consts)


def _tc3_packed(xyzp, gxg, g2, feat, c, d_in, P):
    BN = xyzp.shape[0]
    d2 = c['KW'].shape[1] // K_N
    dout = 2 * d2
    consts = [c[k] for k in _PK_KEYS3]

    def body(xyz_ref, gx_ref, g2_ref, feat_ref, *refs):
        cd = {k: r[...] for k, r in zip(_PK_KEYS3, refs[:-1])}
        out_ref = refs[-1]
        fx = _pk_lfa1(xyz_ref[...], gx_ref[...], cd)
        fx2 = _leaky(cd['g2t'] * (_mm(fx, cd['BDl2']) + cd['b2t']) * BN_SCALE
                     + cd['be2t'])
        f_agg = _pk_att(g2_ref[...], fx2, cd, 'att2')
        f_m2 = _bn(_mm(f_agg, cd['mlp2_W']) + cd['mlp2_b'], cd['mlp2_g'], cd['mlp2_be'])
        sc = _bn(_mm(feat_ref[...], cd['shortcut_W']) + cd['shortcut_b'],
                 cd['shortcut_g'], cd['shortcut_be'])
        out_ref[...] = _leaky(f_m2 + sc)

    return pl.pallas_call(
        body, grid=(BN // P,),
        in_specs=[_row_spec(P, 8), _row_spec(P, K_N * (8 + d2)), _row_spec(P, K_N * d2),
                  _row_spec(P, d_in)]
                 + [_w_full(x.shape) for x in consts],
        out_specs=_row_spec(P, 2 * dout),
        out_shape=jax.ShapeDtypeStruct((BN, 2 * dout), jnp.float32),
    )(xyzp, gxg, g2, feat, *consts)


def _lfa1(xyz, gx, w_dis, w_rel, w_xyz, w_ngx, b1, g1, be1):
    K, P, _ = gx.shape
    xyzb = jnp.broadcast_to(xyz[None], (K, P, 8))
    rel = xyzb - gx
    dis = jnp.sqrt(jnp.sum(rel * rel, axis=-1, keepdims=True) + 1e-12)
    t = (_mm3(rel, w_rel) + _mm3(xyzb, w_xyz) + _mm3(gx, w_ngx)
         + dis * w_dis[0][None, None, :] + b1)
    return _leaky(_bn(t, g1, be1))


def _att_pool(fa, fb, wfc_a, wfc_b, bfc, wam_a, wam_b, bam, gam, beam, d2):
    A = _mm3(fa, wfc_a) + _mm3(fb, wfc_b) + bfc
    A = A - jnp.max(A, axis=0, keepdims=True)
    E = jnp.exp(A)
    Ssc = E / jnp.sum(E, axis=0, keepdims=True)
    agg_lo = jnp.sum(fa * Ssc[..., :d2], axis=0)
    agg_hi = jnp.sum(fb * Ssc[..., d2:], axis=0)
    return _leaky(_bn(_mm(agg_lo, wam_a) + _mm(agg_hi, wam_b) + bam, gam, beam))


def _tc2(xyzp, gxg, wd, P):
    BN = xyzp.shape[0]
    d2 = wd['lfa1_rel'].shape[1]

    def body(xyz_ref, gxg_ref, w_dis, w_rel, w_xyz, w_ngx, b_l1, g_l1, be_l1,
             wfc_a, wfc_b, bfc, wam_a, wam_b, bam, gam, beam, out_ref):
        gxg = gxg_ref[...]
        gx = gxg[..., :8]
        g1 = gxg[..., 8:]
        fx = _lfa1(xyz_ref[...], gx, w_dis[...], w_rel[...], w_xyz[...],
                   w_ngx[...], b_l1[...], g_l1[...], be_l1[...])
        out_ref[...] = _att_pool(g1, fx, wfc_a[...], wfc_b[...], bfc[...],
                                 wam_a[...], wam_b[...], bam[...], gam[...], beam[...], d2)

    dout = 2 * d2
    return pl.pallas_call(
        body, grid=(BN // P,),
        in_specs=[_row_spec(P, 8), _kp_spec(P, 8 + d2),
                  _w_full((1, d2)), _w_full((8, d2)), _w_full((8, d2)), _w_full((8, d2)),
                  _w_full((1, d2)), _w_full((1, d2)), _w_full((1, d2)),
                  _w_full((d2, dout)), _w_full((d2, dout)), _w_full((1, dout)),
                  _w_full((d2, d2)), _w_full((d2, d2)), _w_full((1, d2)),
                  _w_full((1, d2)), _w_full((1, d2))],
        out_specs=_row_spec(P, d2),
        out_shape=jax.ShapeDtypeStruct((BN, d2), jnp.float32),
    )(xyzp, gxg,
      wd['lfa1_dis'], wd['lfa1_rel'], wd['lfa1_xyz'], wd['lfa1_ngx'],
      wd['lfa1_b'], wd['lfa1_g'], wd['lfa1_be'],
      wd['att1_fc_a'], wd['att1_fc_b'], wd['att1_fc_bias'],
      wd['att1_mlp_a'], wd['att1_mlp_b'], wd['att1_mlp_bias'],
      wd['att1_mlp_g'], wd['att1_mlp_be'])


def _tc3(xyzp, gxg, g2, feat, wd, d_in, P):
    BN = xyzp.shape[0]
    d2 = wd['lfa1_rel'].shape[1]
    dout = 2 * d2

    def body(xyz_ref, gx_ref, g2_ref, feat_ref,
             w_dis, w_rel, w_xyz, w_ngx, b_l1, g_l1, be_l1,
             w_l2, b_l2, g_l2, be_l2,
             wfc_a, wfc_b, bfc, wam_a, wam_b, bam, gam, beam,
             w_m2, b_m2, g_m2, be_m2, w_sc, b_sc, g_sc, be_sc, out_ref):
        fx = _lfa1(xyz_ref[...], gx_ref[..., :8], w_dis[...], w_rel[...], w_xyz[...],
                   w_ngx[...], b_l1[...], g_l1[...], be_l1[...])
        fx2 = _leaky(_bn(_mm3(fx, w_l2[...]) + b_l2[...], g_l2[...], be_l2[...]))
        f_agg = _att_pool(g2_ref[...], fx2, wfc_a[...], wfc_b[...], bfc[...],
                          wam_a[...], wam_b[...], bam[...], gam[...], beam[...], d2)
        f_m2 = _bn(_mm(f_agg, w_m2[...]) + b_m2[...], g_m2[...], be_m2[...])
        sc = _bn(_mm(feat_ref[...], w_sc[...]) + b_sc[...], g_sc[...], be_sc[...])
        out_ref[...] = _leaky(f_m2 + sc)

    return pl.pallas_call(
        body, grid=(BN // P,),
        in_specs=[_row_spec(P, 8), _kp_spec(P, 8 + d2), _kp_spec(P, d2), _row_spec(P, d_in),
                  _w_full((1, d2)), _w_full((8, d2)), _w_full((8, d2)), _w_full((8, d2)),
                  _w_full((1, d2)), _w_full((1, d2)), _w_full((1, d2)),
                  _w_full((d2, d2)), _w_full((1, d2)), _w_full((1, d2)), _w_full((1, d2)),
                  _w_full((d2, dout)), _w_full((d2, dout)), _w_full((1, dout)),
                  _w_full((d2, dout)), _w_full((d2, dout)), _w_full((1, dout)),
                  _w_full((1, dout)), _w_full((1, dout)),
                  _w_full((dout, 2 * dout)), _w_full((1, 2 * dout)),
                  _w_full((1, 2 * dout)), _w_full((1, 2 * dout)),
                  _w_full((d_in, 2 * dout)), _w_full((1, 2 * dout)),
                  _w_full((1, 2 * dout)), _w_full((1, 2 * dout))],
        out_specs=_row_spec(P, 2 * dout),
        out_shape=jax.ShapeDtypeStruct((BN, 2 * dout), jnp.float32),
    )(xyzp, gxg, g2, feat,
      wd['lfa1_dis'], wd['lfa1_rel'], wd['lfa1_xyz'], wd['lfa1_ngx'],
      wd['lfa1_b'], wd['lfa1_g'], wd['lfa1_be'],
      wd['lfa2_W'], wd['lfa2_b'], wd['lfa2_g'], wd['lfa2_be'],
      wd['att2_fc_a'], wd['att2_fc_b'], wd['att2_fc_bias'],
      wd['att2_mlp_a'], wd['att2_mlp_b'], wd['att2_mlp_bias'],
      wd['att2_mlp_g'], wd['att2_mlp_be'],
      wd['mlp2_W'], wd['mlp2_b'], wd['mlp2_g'], wd['mlp2_be'],
      wd['sc_W'], wd['sc_b'], wd['sc_g'], wd['sc_be'])


def _tc_dec0(pg, p, P=64):
    BN = pg.shape[0]

    def body(pg_ref, w, b, g, be, out_ref):
        feat = _halvmax(pg_ref[...], 512)
        out_ref[...] = _leaky(_bn(_mm(feat, w[...]) + b[...], g[...], be[...]))

    return pl.pallas_call(
        body, grid=(BN // P,),
        in_specs=[_row_spec(P, K_N * 512),
                  _w_full((512, 512)), _w_full((1, 512)), _w_full((1, 512)), _w_full((1, 512))],
        out_specs=_row_spec(P, 512),
        out_shape=jax.ShapeDtypeStruct((BN, 512), jnp.float32),
    )(pg, p['decoder_0_W'], p['decoder_0_b'], p['decoder_0_gamma'], p['decoder_0_beta'])


def _tc_dec(skip, fi, wa, wb, bias, g, be, P):
    BN, ds = skip.shape
    df = fi.shape[1]
    do = wa.shape[1]

    def body(s_ref, f_ref, wa_r, wb_r, b_r, g_r, be_r, out_ref):
        y = _mm(s_ref[...], wa_r[...]) + _mm(f_ref[...], wb_r[...]) + b_r[...]
        out_ref[...] = _leaky(_bn(y, g_r[...], be_r[...]))

    return pl.pallas_call(
        body, grid=(BN // P,),
        in_specs=[_row_spec(P, ds), _row_spec(P, df),
                  _w_full((ds, do)), _w_full((df, do)), _w_full((1, do)),
                  _w_full((1, do)), _w_full((1, do))],
        out_specs=_row_spec(P, do),
        out_shape=jax.ShapeDtypeStruct((BN, do), jnp.float32),
    )(skip, fi, wa, wb, bias, g, be)


def _tc_head(p, skip, fi, P=2048):
    BN = skip.shape[0]

    def body(s_ref, f_ref, w3a, w3b, b3, g3, be3, w_f1, b_f1, g1, be1,
             w_f2, b_f2, g2, be2, w_fc, b_fc, out_ref):
        y = _mm(s_ref[...], w3a[...]) + _mm(f_ref[...], w3b[...]) + b3[...]
        y = _leaky(_bn(y, g3[...], be3[...]))
        y = _leaky(_bn(_mm(y, w_f1[...]) + b_f1[...], g1[...], be1[...]))
        y = _leaky(_bn(_mm(y, w_f2[...]) + b_f2[...], g2[...], be2[...]))
        out_ref[...] = _mm(y, w_fc[...]) + b_fc[...]

    return pl.pallas_call(
        body, grid=(BN // P,),
        in_specs=[_row_spec(P, 32), _row_spec(P, 32),
                  _w_full((32, 32)), _w_full((32, 32)), _w_full((1, 32)),
                  _w_full((1, 32)), _w_full((1, 32)),
                  _w_full((32, 64)), _w_full((1, 64)), _w_full((1, 64)), _w_full((1, 64)),
                  _w_full((64, 32)), _w_full((1, 32)), _w_full((1, 32)), _w_full((1, 32)),
                  _w_full((32, NUM_CLASSES)), _w_full((1, NUM_CLASSES))],
        out_specs=_row_spec(P, NUM_CLASSES),
        out_shape=jax.ShapeDtypeStruct((BN, NUM_CLASSES), jnp.float32),
    )(skip, fi,
      p['dec3_Wa'], p['dec3_Wb'], p['dec3_b2'], p['dec3_gamma2'], p['dec3_beta2'],
      p['fc1_W'], p['fc1_b2'], p['fc1_gamma2'], p['fc1_beta2'],
      p['fc2_W'], p['fc2_b2'], p['fc2_gamma2'], p['fc2_beta2'],
      p['fc_W'], p['fc_b2'])


def _prep_layer(p, i):
    nm = 'enc%d_' % i
    d2 = D_OUT[i] // 2
    r2 = lambda a: a.reshape(1, -1)
    w10 = p[nm + 'lfa_mlp1_W']
    pad = jnp.zeros((1, d2), jnp.float32)
    wd = {
        'lfa1_dis': w10[0:1],
        'lfa1_rel': jnp.concatenate([w10[1:4]] + [pad] * 5, axis=0),
        'lfa1_xyz': jnp.concatenate([w10[4:7]] + [pad] * 5, axis=0),
        'lfa1_ngx': jnp.concatenate([w10[7:10]] + [pad] * 5, axis=0),
        'lfa1_b': r2(p[nm + 'lfa_mlp1_b']),
        'lfa1_g': r2(p[nm + 'lfa_mlp1_gamma']),
        'lfa1_be': r2(p[nm + 'lfa_mlp1_beta']),
        'att1_fc_a': p[nm + 'att1_fc_W'][:d2],
        'att1_fc_b': p[nm + 'att1_fc_W'][d2:],
        'att1_fc_bias': r2(p[nm + 'att1_fc_b']),
        'att1_mlp_a': p[nm + 'att1_mlp_W'][:d2],
        'att1_mlp_b': p[nm + 'att1_mlp_W'][d2:],
        'att1_mlp_bias': r2(p[nm + 'att1_mlp_b']),
        'att1_mlp_g': r2(p[nm + 'att1_mlp_gamma']),
        'att1_mlp_be': r2(p[nm + 'att1_mlp_beta']),
        'lfa2_W': p[nm + 'lfa_mlp2_W'],
        'lfa2_b': r2(p[nm + 'lfa_mlp2_b']),
        'lfa2_g': r2(p[nm + 'lfa_mlp2_gamma']),
        'lfa2_be': r2(p[nm + 'lfa_mlp2_beta']),
        'att2_fc_a': p[nm + 'att2_fc_W'][:d2],
        'att2_fc_b': p[nm + 'att2_fc_W'][d2:],
        'att2_fc_bias': r2(p[nm + 'att2_fc_b']),
        'att2_mlp_a': p[nm + 'att2_mlp_W'][:d2],
        'att2_mlp_b': p[nm + 'att2_mlp_W'][d2:],
        'att2_mlp_bias': r2(p[nm + 'att2_mlp_b']),
        'att2_mlp_g': r2(p[nm + 'att2_mlp_gamma']),
        'att2_mlp_be': r2(p[nm + 'att2_mlp_beta']),
        'mlp2_W': p[nm + 'mlp2_W'],
        'mlp2_b': r2(p[nm + 'mlp2_b']),
        'mlp2_g': r2(p[nm + 'mlp2_gamma']),
        'mlp2_be': r2(p[nm + 'mlp2_beta']),
        'sc_W': p[nm + 'shortcut_W'],
        'sc_b': r2(p[nm + 'shortcut_b']),
        'sc_g': r2(p[nm + 'shortcut_gamma']),
        'sc_be': r2(p[nm + 'shortcut_beta']),
    }
    return wd


def _prep_global(p):
    r2 = lambda a: a.reshape(1, -1)
    q = dict(p)
    q['fc0_b'] = r2(p['fc0_b'])
    q['bn0_gamma'] = r2(p['bn0_gamma'])
    q['bn0_beta'] = r2(p['bn0_beta'])
    for i in range(NUM_LAYERS):
        nm = 'enc%d_mlp1' % i
        q[nm + '_b'] = r2(p[nm + '_b'])
        q[nm + '_gamma'] = r2(p[nm + '_gamma'])
        q[nm + '_beta'] = r2(p[nm + '_beta'])
    q['decoder_0_b'] = r2(p['decoder_0_b'])
    q['decoder_0_gamma'] = r2(p['decoder_0_gamma'])
    q['decoder_0_beta'] = r2(p['decoder_0_beta'])
    for j in range(NUM_LAYERS):
        nm = 'dec%d' % j
        q[nm + '_b2'] = r2(p[nm + '_b'])
        q[nm + '_gamma2'] = r2(p[nm + '_gamma'])
        q[nm + '_beta2'] = r2(p[nm + '_beta'])
    q['dec3_Wa'] = p['dec3_W'][:32]
    q['dec3_Wb'] = p['dec3_W'][32:]
    for nm in ('fc1', 'fc2'):
        q[nm + '_b2'] = r2(p[nm + '_b'])
        q[nm + '_gamma2'] = r2(p[nm + '_gamma'])
        q[nm + '_beta2'] = r2(p[nm + '_beta'])
    q['fc_b2'] = r2(p['fc_b'])
    return q


_P_TC1 = [2048, 512, 256, 128]
_P_PK = [2048, 1024]
_P_KM = [512, 512, 256, 128]


def kernel(params, features,
           xyz0, xyz1, xyz2, xyz3,
           neigh_idx0, neigh_idx1, neigh_idx2, neigh_idx3,
           sub_idx0, sub_idx1, sub_idx2, sub_idx3,
           interp_idx0, interp_idx1, interp_idx2, interp_idx3):
    p = _prep_global(params)
    xyz_list = [xyz0, xyz1, xyz2, xyz3]
    neigh_list = [neigh_idx0, neigh_idx1, neigh_idx2, neigh_idx3]
    sub_list = [sub_idx0, sub_idx1, sub_idx2, sub_idx3]
    interp_list = [interp_idx0, interp_idx1, interp_idx2, interp_idx3]

    xyzp = [jnp.concatenate([x, jnp.zeros(x.shape[:2] + (5,), jnp.float32)],
                            axis=-1).reshape(-1, 8) for x in xyz_list]
    nt_pm = [ni.reshape(-1) for ni in neigh_list]
    nt_km = [jnp.transpose(ni, (2, 0, 1)).reshape(-1) for ni in neigh_list]
    st_pm = [si.reshape(-1) for si in sub_list]
    it = [ii.reshape(-1) for ii in interp_list]

    f0, xf = _tc1_l0(features.reshape(-1, 6), xyzp[0], p)
    feat = f0
    fe0 = None
    skips = []
    pg = None
    for i in range(NUM_LAYERS):
        N = NS[i]
        BN = B * N
        d2 = D_OUT[i] // 2
        q = 8 + d2
        if i < 2:
            c = _packed_consts(params, i)
            gxg = _sc_gather(xf, nt_pm[i], seg=N * K_N, n_tab=N
                             ).reshape(BN, K_N * q)
            fagg1 = _tc2_packed(xyzp[i], gxg, c, _P_PK[i])
            g2 = _sc_gather(fagg1, nt_pm[i], seg=N * K_N, n_tab=N
                            ).reshape(BN, K_N * d2)
            fe = _tc3_packed(xyzp[i], gxg, g2, feat, c, feat.shape[1], _P_PK[i])
        else:
            wd = _prep_layer(params, i)
            gxg = _sc_gather(xf, nt_km[i], seg=N, n_tab=N).reshape(K_N, BN, q)
            fagg1 = _tc2(xyzp[i], gxg, wd, _P_KM[i])
            g2 = _sc_gather(fagg1, nt_km[i], seg=N, n_tab=N).reshape(K_N, BN, d2)
            fe = _tc3(xyzp[i], gxg, g2, feat, wd, feat.shape[1], _P_KM[i])
        if i == 0:
            fe0 = fe
        N1 = NS[i + 1]
        dprev = 2 * D_OUT[i]
        pg = _sc_gather(fe, st_pm[i], seg=N1 * K_N, n_tab=N
                        ).reshape(B * N1, K_N * dprev)
        if i < NUM_LAYERS - 1:
            feat, xf = _tc1(pg, xyzp[i + 1], p, 'enc%d_' % (i + 1),
                            dprev, D_OUT[i + 1] // 2, _P_TC1[i + 1])
            skips.append(feat)

    f = _tc_dec0(pg, p)
    dec_P = [512, 512, 1024]
    for j in range(NUM_LAYERS - 1):
        lev = NUM_LAYERS - 1 - j
        fi = _sc_gather(f, it[lev], seg=NS[lev], n_tab=NS[lev + 1])
        skip = skips[2 - j]
        nm = 'dec%d' % j
        ds = skip.shape[1]
        f = _tc_dec(skip, fi, params[nm + '_W'][:ds], params[nm + '_W'][ds:],
                    p[nm + '_b2'], p[nm + '_gamma2'], p[nm + '_beta2'], dec_P[j])

    fi = _sc_gather(f, it[0], seg=NS[0], n_tab=NS[1])
    out = _tc_head(p, fe0, fi)
    return out.reshape(B, N0, NUM_CLASSES)


# speedup vs baseline: 1.0835x; 1.0835x over previous
import functools

import jax
import jax.numpy as jnp
import numpy as np
from jax import lax
from jax.experimental import pallas as pl
from jax.experimental.pallas import tpu as pltpu
from jax.experimental.pallas import tpu_sc as plsc

NUM_LAYERS = 4
D_OUT = [16, 64, 128, 256]
D_FEATURE = 8
NUM_CLASSES = 13
K_N = 16
B = 2
N0 = 65536
SUB = 4
NS = [N0 // (SUB ** i) for i in range(NUM_LAYERS + 1)]
BN_SCALE = 1.0 / np.sqrt(1.0 + 1e-6)
NW = 32


def _leaky(x):
    return jnp.where(x >= 0, x, 0.2 * x)


@functools.lru_cache(maxsize=None)
def _make_sc_gather(R, d, M, seg, n_tab):
    chunk = M // NW
    assert M % NW == 0 and seg % chunk == 0
    d_eff = max(8, d)
    S = min(chunk, 8192)
    while S * (d_eff + 1) > 98304:
        S //= 2
    n_sub = chunk // S
    mesh = plsc.VectorSubcoreMesh(core_axis_name="c", subcore_axis_name="s")

    @functools.partial(
        pl.kernel, mesh=mesh,
        out_type=jax.ShapeDtypeStruct((M, d), jnp.float32),
        scratch_types=[pltpu.VMEM((S,), jnp.int32),
                       pltpu.VMEM((S, d), jnp.float32),
                       pltpu.SemaphoreType.DMA],
        compiler_params=pltpu.CompilerParams(use_tc_tiling_on_sc=False))
    def k(table_hbm, idx_hbm, out_hbm, idx_v, rows_v, sem):
        c = lax.axis_index("c")
        s = lax.axis_index("s")
        wid = s * 2 + c
        base = wid * chunk
        boff = ((base // seg) % B) * n_tab

        def body(j, _):
            off = base + j * S
            pltpu.sync_copy(idx_hbm.at[pl.ds(off, S)], idx_v)

            def addb(t, _):
                sl = pl.ds(t * 16, 16)
                idx_v[sl] = idx_v[sl] + boff
                return 0

            lax.fori_loop(0, S // 16, addb, 0, unroll=8)
            pltpu.async_copy(table_hbm.at[idx_v], rows_v, sem).wait()
            pltpu.sync_copy(rows_v, out_hbm.at[pl.ds(off, S)])
            return 0

        if n_sub == 1:
            body(0, 0)
        else:
            lax.fori_loop(0, n_sub, body, 0)

    return k


def _sc_gather(table, idx, seg, n_tab):
    R, d = table.shape
    (M,) = idx.shape
    return _make_sc_gather(R, d, M, seg, n_tab)(table, idx)


def _bn(y, g, b):
    return g * y * BN_SCALE + b


def _mm(x, w):
    return jax.lax.dot_general(x, w, (((x.ndim - 1,), (0,)), ((), ())),
                               preferred_element_type=jnp.float32)


def _mm3(x, w):
    K, P, a = x.shape
    return _mm(x.reshape(K * P, a), w).reshape(K, P, w.shape[1])


def _w_full(shp):
    return pl.BlockSpec(shp, lambda *_: tuple(0 for _ in shp))


def _row_spec(P, d):
    return pl.BlockSpec((P, d), lambda i: (i, 0))


def _kp_spec(P, d):
    return pl.BlockSpec((K_N, P, d), lambda i: (0, i, 0))


def _halvmax(x, d_target):
    while x.shape[1] > d_target:
        w = x.shape[1] // 2
        x = jnp.maximum(x[:, :w], x[:, w:])
    return x


def _tc1_l0(feats, p, P=2048):
    BN = feats.shape[0]

    def body(x_ref, w0, b0, g0, be0, w1, b1, g1, be1, f0_ref, fpc_ref):
        y = _leaky(_bn(_mm(x_ref[...], w0[...]) + b0[...], g0[...], be0[...]))
        f0_ref[...] = y
        fpc_ref[...] = _leaky(_bn(_mm(y, w1[...]) + b1[...], g1[...], be1[...]))

    return pl.pallas_call(
        body, grid=(BN // P,),
        in_specs=[_row_spec(P, 6),
                  _w_full((6, 8)), _w_full((1, 8)), _w_full((1, 8)), _w_full((1, 8)),
                  _w_full((8, 8)), _w_full((1, 8)), _w_full((1, 8)), _w_full((1, 8))],
        out_specs=[_row_spec(P, 8), _row_spec(P, 8)],
        out_shape=[jax.ShapeDtypeStruct((BN, 8), jnp.float32),
                   jax.ShapeDtypeStruct((BN, 8), jnp.float32)],
    )(feats, p['fc0_W'], p['fc0_b'], p['bn0_gamma'], p['bn0_beta'],
      p['enc0_mlp1_W'], p['enc0_mlp1_b'], p['enc0_mlp1_gamma'], p['enc0_mlp1_beta'])


def _tc1(pg, p, nm, d_in, d2, P):
    BN = pg.shape[0]

    def body(pg_ref, w1, b1, g1, be1, feat_ref, fpc_ref):
        feat = _halvmax(pg_ref[...], d_in)
        feat_ref[...] = feat
        fpc_ref[...] = _leaky(_bn(_mm(feat, w1[...]) + b1[...], g1[...], be1[...]))

    return pl.pallas_call(
        body, grid=(BN // P,),
        in_specs=[_row_spec(P, K_N * d_in),
                  _w_full((d_in, d2)), _w_full((1, d2)), _w_full((1, d2)), _w_full((1, d2))],
        out_specs=[_row_spec(P, d_in), _row_spec(P, d2)],
        out_shape=[jax.ShapeDtypeStruct((BN, d_in), jnp.float32),
                   jax.ShapeDtypeStruct((BN, d2), jnp.float32)],
    )(pg, p[nm + 'mlp1_W'], p[nm + 'mlp1_b'], p[nm + 'mlp1_gamma'], p[nm + 'mlp1_beta'])


def _packed_consts(params, i):
    nm = 'enc%d_' % i
    d2 = D_OUT[i] // 2
    dout = D_OUT[i]
    K = K_N
    I_K = np.eye(K, dtype=np.float32)
    ones_K = np.ones((K, 1), dtype=np.float32)
    eye8 = np.eye(8, dtype=np.float32)
    Id2 = np.eye(d2, dtype=np.float32)
    Ido = np.eye(dout, dtype=np.float32)
    kron = jnp.kron
    tile = lambda a: jnp.tile(a.reshape(1, -1), (1, K))

    w10 = params[nm + 'lfa_mlp1_W']
    pad5 = jnp.zeros((5, d2), jnp.float32)
    wdis = w10[0:1]
    wrel8 = jnp.concatenate([w10[1:4], pad5], axis=0)
    wxyz8 = jnp.concatenate([w10[4:7], pad5], axis=0)
    wngx8 = jnp.concatenate([w10[7:10], pad5], axis=0)

    c = {}
    c['T8'] = jnp.asarray(np.tile(eye8, (1, K)))
    c['G8'] = jnp.asarray(np.kron(I_K, np.ones((8, 1), np.float32)))
    c['Wxc'] = jnp.tile(wrel8 + wxyz8, (1, K))
    c['BDgx'] = kron(jnp.asarray(I_K), wngx8 - wrel8)
    c['KW'] = kron(jnp.asarray(I_K), wdis)
    c['b1t'] = tile(params[nm + 'lfa_mlp1_b'])
    c['g1t'] = tile(params[nm + 'lfa_mlp1_gamma'])
    c['be1t'] = tile(params[nm + 'lfa_mlp1_beta'])

    c['Gdo'] = jnp.asarray(np.kron(ones_K, Ido))
    c['Tdo'] = jnp.asarray(np.tile(Ido, (1, K)))
    sel_lo = np.kron(I_K, np.vstack([Id2, np.zeros((d2, d2), np.float32)]))
    sel_hi = np.kron(I_K, np.vstack([np.zeros((d2, d2), np.float32), Id2]))
    c['SEL_lo'] = jnp.asarray(sel_lo)
    c['SEL_hi'] = jnp.asarray(sel_hi)

    for att, od in (('att1', d2), ('att2', dout)):
        W = params[nm + att + '_fc_W']
        c[att + '_BDfc_g'] = kron(jnp.asarray(I_K), W[:d2])
        c[att + '_BDfc_x'] = kron(jnp.asarray(I_K), W[d2:])
        c[att + '_bfct'] = tile(params[nm + att + '_fc_b'])
        Wm = params[nm + att + '_mlp_W']
        c[att + '_GWam_lo'] = kron(jnp.asarray(ones_K), Wm[:d2])
        c[att + '_GWam_hi'] = kron(jnp.asarray(ones_K), Wm[d2:])
        r2 = lambda a: a.reshape(1, -1)
        c[att + '_bam'] = r2(params[nm + att + '_mlp_b'])
        c[att + '_gam'] = r2(params[nm + att + '_mlp_gamma'])
        c[att + '_beam'] = r2(params[nm + att + '_mlp_beta'])

    c['BDl2'] = kron(jnp.asarray(I_K), params[nm + 'lfa_mlp2_W'])
    c['b2t'] = tile(params[nm + 'lfa_mlp2_b'])
    c['g2t'] = tile(params[nm + 'lfa_mlp2_gamma'])
    c['be2t'] = tile(params[nm + 'lfa_mlp2_beta'])

    r2 = lambda a: a.reshape(1, -1)
    for s in ('mlp2', 'shortcut'):
        c[s + '_W'] = params[nm + s + '_W']
        c[s + '_b'] = r2(params[nm + s + '_b'])
        c[s + '_g'] = r2(params[nm + s + '_gamma'])
        c[s + '_be'] = r2(params[nm + s + '_beta'])
    return c


def _pk_lfa1(xyz, gx, c):
    xyzt = _mm(xyz, c['T8'])
    rel = xyzt - gx
    dis = jnp.sqrt(_mm(rel * rel, c['G8']) + 1e-12)
    t = _mm(xyz, c['Wxc']) + _mm(gx, c['BDgx']) + _mm(dis, c['KW']) + c['b1t']
    return _leaky(c['g1t'] * t * BN_SCALE + c['be1t'])


def _pk_att(fa, fb, c, att):
    A = _mm(fa, c[att + '_BDfc_g']) + _mm(fb, c[att + '_BDfc_x']) + c[att + '_bfct']
    A = A - jnp.max(A, axis=-1, keepdims=True)
    E = jnp.exp(A)
    rden = 1.0 / _mm(E, c['Gdo'])
    S = E * _mm(rden, c['Tdo'])
    agg = (_mm(fa * _mm(S, c['SEL_lo']), c[att + '_GWam_lo'])
           + _mm(fb * _mm(S, c['SEL_hi']), c[att + '_GWam_hi']))
    return _leaky(_bn(agg + c[att + '_bam'], c[att + '_gam'], c[att + '_beam']))


_PK_KEYS2 = ['T8', 'G8', 'Wxc', 'BDgx', 'KW', 'b1t', 'g1t', 'be1t',
             'Gdo', 'Tdo', 'SEL_lo', 'SEL_hi',
             'att1_BDfc_g', 'att1_BDfc_x', 'att1_bfct',
             'att1_GWam_lo', 'att1_GWam_hi', 'att1_bam', 'att1_gam', 'att1_beam']

_PK_KEYS3 = ['T8', 'G8', 'Wxc', 'BDgx', 'KW', 'b1t', 'g1t', 'be1t',
             'BDl2', 'b2t', 'g2t', 'be2t',
             'Gdo', 'Tdo', 'SEL_lo', 'SEL_hi',
             'att2_BDfc_g', 'att2_BDfc_x', 'att2_bfct',
             'att2_GWam_lo', 'att2_GWam_hi', 'att2_bam', 'att2_gam', 'att2_beam',
             'mlp2_W', 'mlp2_b', 'mlp2_g', 'mlp2_be',
             'shortcut_W', 'shortcut_b', 'shortcut_g', 'shortcut_be']


def _tc2_packed(xyzp, gx, g1, c, P):
    BN = xyzp.shape[0]
    d2 = c['KW'].shape[1] // K_N
    consts = [c[k] for k in _PK_KEYS2]

    def body(xyz_ref, gx_ref, g1_ref, *refs):
        cd = {k: r[...] for k, r in zip(_PK_KEYS2, refs[:-1])}
        out_ref = refs[-1]
        fx = _pk_lfa1(xyz_ref[...], gx_ref[...], cd)
        out_ref[...] = _pk_att(g1_ref[...], fx, cd, 'att1')

    return pl.pallas_call(
        body, grid=(BN // P,),
        in_specs=[_row_spec(P, 8), _row_spec(P, 128), _row_spec(P, K_N * d2)]
                 + [_w_full(x.shape) for x in consts],
        out_specs=_row_spec(P, d2),
        out_shape=jax.ShapeDtypeStruct((BN, d2), jnp.float32),
    )(xyzp, gx, g1, *consts)


def _tc3_packed(xyzp, gx, g2, feat, c, d_in, P):
    BN = xyzp.shape[0]
    d2 = c['KW'].shape[1] // K_N
    dout = 2 * d2
    consts = [c[k] for k in _PK_KEYS3]

    def body(xyz_ref, gx_ref, g2_ref, feat_ref, *refs):
        cd = {k: r[...] for k, r in zip(_PK_KEYS3, refs[:-1])}
        out_ref = refs[-1]
        fx = _pk_lfa1(xyz_ref[...], gx_ref[...], cd)
        fx2 = _leaky(cd['g2t'] * (_mm(fx, cd['BDl2']) + cd['b2t']) * BN_SCALE
                     + cd['be2t'])
        f_agg = _pk_att(g2_ref[...], fx2, cd, 'att2')
        f_m2 = _bn(_mm(f_agg, cd['mlp2_W']) + cd['mlp2_b'], cd['mlp2_g'], cd['mlp2_be'])
        sc = _bn(_mm(feat_ref[...], cd['shortcut_W']) + cd['shortcut_b'],
                 cd['shortcut_g'], cd['shortcut_be'])
        out_ref[...] = _leaky(f_m2 + sc)

    return pl.pallas_call(
        body, grid=(BN // P,),
        in_specs=[_row_spec(P, 8), _row_spec(P, 128), _row_spec(P, K_N * d2),
                  _row_spec(P, d_in)]
                 + [_w_full(x.shape) for x in consts],
        out_specs=_row_spec(P, 2 * dout),
        out_shape=jax.ShapeDtypeStruct((BN, 2 * dout), jnp.float32),
    )(xyzp, gx, g2, feat, *consts)


def _lfa1(xyz, gx, w_dis, w_rel, w_xyz, w_ngx, b1, g1, be1):
    K, P, _ = gx.shape
    xyzb = jnp.broadcast_to(xyz[None], (K, P, 8))
    rel = xyzb - gx
    dis = jnp.sqrt(jnp.sum(rel * rel, axis=-1, keepdims=True) + 1e-12)
    t = (_mm3(rel, w_rel) + _mm3(xyzb, w_xyz) + _mm3(gx, w_ngx)
         + dis * w_dis[0][None, None, :] + b1)
    return _leaky(_bn(t, g1, be1))


def _att_pool(fa, fb, wfc_a, wfc_b, bfc, wam_a, wam_b, bam, gam, beam, d2):
    A = _mm3(fa, wfc_a) + _mm3(fb, wfc_b) + bfc
    A = A - jnp.max(A, axis=0, keepdims=True)
    E = jnp.exp(A)
    Ssc = E / jnp.sum(E, axis=0, keepdims=True)
    agg_lo = jnp.sum(fa * Ssc[..., :d2], axis=0)
    agg_hi = jnp.sum(fb * Ssc[..., d2:], axis=0)
    return _leaky(_bn(_mm(agg_lo, wam_a) + _mm(agg_hi, wam_b) + bam, gam, beam))


def _tc2(xyzp, gx, g1, wd, P):
    BN = xyzp.shape[0]
    d2 = wd['lfa1_rel'].shape[1]

    def body(xyz_ref, gx_ref, g1_ref,
             w_dis, w_rel, w_xyz, w_ngx, b_l1, g_l1, be_l1,
             wfc_a, wfc_b, bfc, wam_a, wam_b, bam, gam, beam, out_ref):
        fx = _lfa1(xyz_ref[...], gx_ref[...], w_dis[...], w_rel[...], w_xyz[...],
                   w_ngx[...], b_l1[...], g_l1[...], be_l1[...])
        out_ref[...] = _att_pool(g1_ref[...], fx, wfc_a[...], wfc_b[...], bfc[...],
                                 wam_a[...], wam_b[...], bam[...], gam[...], beam[...], d2)

    dout = 2 * d2
    return pl.pallas_call(
        body, grid=(BN // P,),
        in_specs=[_row_spec(P, 8), _kp_spec(P, 8), _kp_spec(P, d2),
                  _w_full((1, d2)), _w_full((8, d2)), _w_full((8, d2)), _w_full((8, d2)),
                  _w_full((1, d2)), _w_full((1, d2)), _w_full((1, d2)),
                  _w_full((d2, dout)), _w_full((d2, dout)), _w_full((1, dout)),
                  _w_full((d2, d2)), _w_full((d2, d2)), _w_full((1, d2)),
                  _w_full((1, d2)), _w_full((1, d2))],
        out_specs=_row_spec(P, d2),
        out_shape=jax.ShapeDtypeStruct((BN, d2), jnp.float32),
    )(xyzp, gx, g1,
      wd['lfa1_dis'], wd['lfa1_rel'], wd['lfa1_xyz'], wd['lfa1_ngx'],
      wd['lfa1_b'], wd['lfa1_g'], wd['lfa1_be'],
      wd['att1_fc_a'], wd['att1_fc_b'], wd['att1_fc_bias'],
      wd['att1_mlp_a'], wd['att1_mlp_b'], wd['att1_mlp_bias'],
      wd['att1_mlp_g'], wd['att1_mlp_be'])


def _tc3(xyzp, gx, g2, feat, wd, d_in, P):
    BN = xyzp.shape[0]
    d2 = wd['lfa1_rel'].shape[1]
    dout = 2 * d2

    def body(xyz_ref, gx_ref, g2_ref, feat_ref,
             w_dis, w_rel, w_xyz, w_ngx, b_l1, g_l1, be_l1,
             w_l2, b_l2, g_l2, be_l2,
             wfc_a, wfc_b, bfc, wam_a, wam_b, bam, gam, beam,
             w_m2, b_m2, g_m2, be_m2, w_sc, b_sc, g_sc, be_sc, out_ref):
        fx = _lfa1(xyz_ref[...], gx_ref[...], w_dis[...], w_rel[...], w_xyz[...],
                   w_ngx[...], b_l1[...], g_l1[...], be_l1[...])
        fx2 = _leaky(_bn(_mm3(fx, w_l2[...]) + b_l2[...], g_l2[...], be_l2[...]))
        f_agg = _att_pool(g2_ref[...], fx2, wfc_a[...], wfc_b[...], bfc[...],
                          wam_a[...], wam_b[...], bam[...], gam[...], beam[...], d2)
        f_m2 = _bn(_mm(f_agg, w_m2[...]) + b_m2[...], g_m2[...], be_m2[...])
        sc = _bn(_mm(feat_ref[...], w_sc[...]) + b_sc[...], g_sc[...], be_sc[...])
        out_ref[...] = _leaky(f_m2 + sc)

    return pl.pallas_call(
        body, grid=(BN // P,),
        in_specs=[_row_spec(P, 8), _kp_spec(P, 8), _kp_spec(P, d2), _row_spec(P, d_in),
                  _w_full((1, d2)), _w_full((8, d2)), _w_full((8, d2)), _w_full((8, d2)),
                  _w_full((1, d2)), _w_full((1, d2)), _w_full((1, d2)),
                  _w_full((d2, d2)), _w_full((1, d2)), _w_full((1, d2)), _w_full((1, d2)),
                  _w_full((d2, dout)), _w_full((d2, dout)), _w_full((1, dout)),
                  _w_full((d2, dout)), _w_full((d2, dout)), _w_full((1, dout)),
                  _w_full((1, dout)), _w_full((1, dout)),
                  _w_full((dout, 2 * dout)), _w_full((1, 2 * dout)),
                  _w_full((1, 2 * dout)), _w_full((1, 2 * dout)),
                  _w_full((d_in, 2 * dout)), _w_full((1, 2 * dout)),
                  _w_full((1, 2 * dout)), _w_full((1, 2 * dout))],
        out_specs=_row_spec(P, 2 * dout),
        out_shape=jax.ShapeDtypeStruct((BN, 2 * dout), jnp.float32),
    )(xyzp, gx, g2, feat,
      wd['lfa1_dis'], wd['lfa1_rel'], wd['lfa1_xyz'], wd['lfa1_ngx'],
      wd['lfa1_b'], wd['lfa1_g'], wd['lfa1_be'],
      wd['lfa2_W'], wd['lfa2_b'], wd['lfa2_g'], wd['lfa2_be'],
      wd['att2_fc_a'], wd['att2_fc_b'], wd['att2_fc_bias'],
      wd['att2_mlp_a'], wd['att2_mlp_b'], wd['att2_mlp_bias'],
      wd['att2_mlp_g'], wd['att2_mlp_be'],
      wd['mlp2_W'], wd['mlp2_b'], wd['mlp2_g'], wd['mlp2_be'],
      wd['sc_W'], wd['sc_b'], wd['sc_g'], wd['sc_be'])


def _tc_dec0(pg, p, P=64):
    BN = pg.shape[0]

    def body(pg_ref, w, b, g, be, out_ref):
        feat = _halvmax(pg_ref[...], 512)
        out_ref[...] = _leaky(_bn(_mm(feat, w[...]) + b[...], g[...], be[...]))

    return pl.pallas_call(
        body, grid=(BN // P,),
        in_specs=[_row_spec(P, K_N * 512),
                  _w_full((512, 512)), _w_full((1, 512)), _w_full((1, 512)), _w_full((1, 512))],
        out_specs=_row_spec(P, 512),
        out_shape=jax.ShapeDtypeStruct((BN, 512), jnp.float32),
    )(pg, p['decoder_0_W'], p['decoder_0_b'], p['decoder_0_gamma'], p['decoder_0_beta'])


def _tc_dec(skip, fi, wa, wb, bias, g, be, P):
    BN, ds = skip.shape
    df = fi.shape[1]
    do = wa.shape[1]

    def body(s_ref, f_ref, wa_r, wb_r, b_r, g_r, be_r, out_ref):
        y = _mm(s_ref[...], wa_r[...]) + _mm(f_ref[...], wb_r[...]) + b_r[...]
        out_ref[...] = _leaky(_bn(y, g_r[...], be_r[...]))

    return pl.pallas_call(
        body, grid=(BN // P,),
        in_specs=[_row_spec(P, ds), _row_spec(P, df),
                  _w_full((ds, do)), _w_full((df, do)), _w_full((1, do)),
                  _w_full((1, do)), _w_full((1, do))],
        out_specs=_row_spec(P, do),
        out_shape=jax.ShapeDtypeStruct((BN, do), jnp.float32),
    )(skip, fi, wa, wb, bias, g, be)


def _tc_head(p, skip, fi, P=2048):
    BN = skip.shape[0]

    def body(s_ref, f_ref, w3a, w3b, b3, g3, be3, w_f1, b_f1, g1, be1,
             w_f2, b_f2, g2, be2, w_fc, b_fc, out_ref):
        y = _mm(s_ref[...], w3a[...]) + _mm(f_ref[...], w3b[...]) + b3[...]
        y = _leaky(_bn(y, g3[...], be3[...]))
        y = _leaky(_bn(_mm(y, w_f1[...]) + b_f1[...], g1[...], be1[...]))
        y = _leaky(_bn(_mm(y, w_f2[...]) + b_f2[...], g2[...], be2[...]))
        out_ref[...] = _mm(y, w_fc[...]) + b_fc[...]

    return pl.pallas_call(
        body, grid=(BN // P,),
        in_specs=[_row_spec(P, 32), _row_spec(P, 32),
                  _w_full((32, 32)), _w_full((32, 32)), _w_full((1, 32)),
                  _w_full((1, 32)), _w_full((1, 32)),
                  _w_full((32, 64)), _w_full((1, 64)), _w_full((1, 64)), _w_full((1, 64)),
                  _w_full((64, 32)), _w_full((1, 32)), _w_full((1, 32)), _w_full((1, 32)),
                  _w_full((32, NUM_CLASSES)), _w_full((1, NUM_CLASSES))],
        out_specs=_row_spec(P, NUM_CLASSES),
        out_shape=jax.ShapeDtypeStruct((BN, NUM_CLASSES), jnp.float32),
    )(skip, fi,
      p['dec3_Wa'], p['dec3_Wb'], p['dec3_b2'], p['dec3_gamma2'], p['dec3_beta2'],
      p['fc1_W'], p['fc1_b2'], p['fc1_gamma2'], p['fc1_beta2'],
      p['fc2_W'], p['fc2_b2'], p['fc2_gamma2'], p['fc2_beta2'],
      p['fc_W'], p['fc_b2'])


def _prep_layer(p, i):
    nm = 'enc%d_' % i
    d2 = D_OUT[i] // 2
    r2 = lambda a: a.reshape(1, -1)
    w10 = p[nm + 'lfa_mlp1_W']
    pad = jnp.zeros((1, d2), jnp.float32)
    wd = {
        'lfa1_dis': w10[0:1],
        'lfa1_rel': jnp.concatenate([w10[1:4]] + [pad] * 5, axis=0),
        'lfa1_xyz': jnp.concatenate([w10[4:7]] + [pad] * 5, axis=0),
        'lfa1_ngx': jnp.concatenate([w10[7:10]] + [pad] * 5, axis=0),
        'lfa1_b': r2(p[nm + 'lfa_mlp1_b']),
        'lfa1_g': r2(p[nm + 'lfa_mlp1_gamma']),
        'lfa1_be': r2(p[nm + 'lfa_mlp1_beta']),
        'att1_fc_a': p[nm + 'att1_fc_W'][:d2],
        'att1_fc_b': p[nm + 'att1_fc_W'][d2:],
        'att1_fc_bias': r2(p[nm + 'att1_fc_b']),
        'att1_mlp_a': p[nm + 'att1_mlp_W'][:d2],
        'att1_mlp_b': p[nm + 'att1_mlp_W'][d2:],
        'att1_mlp_bias': r2(p[nm + 'att1_mlp_b']),
        'att1_mlp_g': r2(p[nm + 'att1_mlp_gamma']),
        'att1_mlp_be': r2(p[nm + 'att1_mlp_beta']),
        'lfa2_W': p[nm + 'lfa_mlp2_W'],
        'lfa2_b': r2(p[nm + 'lfa_mlp2_b']),
        'lfa2_g': r2(p[nm + 'lfa_mlp2_gamma']),
        'lfa2_be': r2(p[nm + 'lfa_mlp2_beta']),
        'att2_fc_a': p[nm + 'att2_fc_W'][:d2],
        'att2_fc_b': p[nm + 'att2_fc_W'][d2:],
        'att2_fc_bias': r2(p[nm + 'att2_fc_b']),
        'att2_mlp_a': p[nm + 'att2_mlp_W'][:d2],
        'att2_mlp_b': p[nm + 'att2_mlp_W'][d2:],
        'att2_mlp_bias': r2(p[nm + 'att2_mlp_b']),
        'att2_mlp_g': r2(p[nm + 'att2_mlp_gamma']),
        'att2_mlp_be': r2(p[nm + 'att2_mlp_beta']),
        'mlp2_W': p[nm + 'mlp2_W'],
        'mlp2_b': r2(p[nm + 'mlp2_b']),
        'mlp2_g': r2(p[nm + 'mlp2_gamma']),
        'mlp2_be': r2(p[nm + 'mlp2_beta']),
        'sc_W': p[nm + 'shortcut_W'],
        'sc_b': r2(p[nm + 'shortcut_b']),
        'sc_g': r2(p[nm + 'shortcut_gamma']),
        'sc_be': r2(p[nm + 'shortcut_beta']),
    }
    return wd


def _prep_global(p):
    r2 = lambda a: a.reshape(1, -1)
    q = dict(p)
    q['fc0_b'] = r2(p['fc0_b'])
    q['bn0_gamma'] = r2(p['bn0_gamma'])
    q['bn0_beta'] = r2(p['bn0_beta'])
    for i in range(NUM_LAYERS):
        nm = 'enc%d_mlp1' % i
        q[nm + '_b'] = r2(p[nm + '_b'])
        q[nm + '_gamma'] = r2(p[nm + '_gamma'])
        q[nm + '_beta'] = r2(p[nm + '_beta'])
    q['decoder_0_b'] = r2(p['decoder_0_b'])
    q['decoder_0_gamma'] = r2(p['decoder_0_gamma'])
    q['decoder_0_beta'] = r2(p['decoder_0_beta'])
    for j in range(NUM_LAYERS):
        nm = 'dec%d' % j
        q[nm + '_b2'] = r2(p[nm + '_b'])
        q[nm + '_gamma2'] = r2(p[nm + '_gamma'])
        q[nm + '_beta2'] = r2(p[nm + '_beta'])
    q['dec3_Wa'] = p['dec3_W'][:32]
    q['dec3_Wb'] = p['dec3_W'][32:]
    for nm in ('fc1', 'fc2'):
        q[nm + '_b2'] = r2(p[nm + '_b'])
        q[nm + '_gamma2'] = r2(p[nm + '_gamma'])
        q[nm + '_beta2'] = r2(p[nm + '_beta'])
    q['fc_b2'] = r2(p['fc_b'])
    return q


_P_TC1 = [2048, 512, 256, 128]
_P_PK = [2048, 1024]
_P_KM = [512, 512, 256, 128]


def kernel(params, features,
           xyz0, xyz1, xyz2, xyz3,
           neigh_idx0, neigh_idx1, neigh_idx2, neigh_idx3,
           sub_idx0, sub_idx1, sub_idx2, sub_idx3,
           interp_idx0, interp_idx1, interp_idx2, interp_idx3):
    p = _prep_global(params)
    xyz_list = [xyz0, xyz1, xyz2, xyz3]
    neigh_list = [neigh_idx0, neigh_idx1, neigh_idx2, neigh_idx3]
    sub_list = [sub_idx0, sub_idx1, sub_idx2, sub_idx3]
    interp_list = [interp_idx0, interp_idx1, interp_idx2, interp_idx3]

    xyzp = [jnp.concatenate([x, jnp.zeros(x.shape[:2] + (5,), jnp.float32)],
                            axis=-1).reshape(-1, 8) for x in xyz_list]
    nt_pm = [ni.reshape(-1) for ni in neigh_list]
    nt_km = [jnp.transpose(ni, (2, 0, 1)).reshape(-1) for ni in neigh_list]
    st_pm = [si.reshape(-1) for si in sub_list]
    it = [ii.reshape(-1) for ii in interp_list]

    gx_all = []
    for i in range(NUM_LAYERS):
        N = NS[i]
        if i < 2:
            g = _sc_gather(xyzp[i], nt_pm[i], seg=N * K_N, n_tab=N
                           ).reshape(B * N, K_N * 8)
        else:
            g = _sc_gather(xyzp[i], nt_km[i], seg=N, n_tab=N
                           ).reshape(K_N, B * N, 8)
        gx_all.append(g)

    f0, fpc = _tc1_l0(features.reshape(-1, 6), p)
    feat = f0
    fe0 = None
    skips = []
    pg = None
    for i in range(NUM_LAYERS):
        N = NS[i]
        BN = B * N
        d2 = D_OUT[i] // 2
        if i < 2:
            c = _packed_consts(params, i)
            gx = gx_all[i]
            g1 = _sc_gather(fpc, nt_pm[i], seg=N * K_N, n_tab=N
                            ).reshape(BN, K_N * d2)
            fagg1 = _tc2_packed(xyzp[i], gx, g1, c, _P_PK[i])
            g2 = _sc_gather(fagg1, nt_pm[i], seg=N * K_N, n_tab=N
                            ).reshape(BN, K_N * d2)
            fe = _tc3_packed(xyzp[i], gx, g2, feat, c, feat.shape[1], _P_PK[i])
        else:
            wd = _prep_layer(params, i)
            gx = gx_all[i]
            g1 = _sc_gather(fpc, nt_km[i], seg=N, n_tab=N).reshape(K_N, BN, d2)
            fagg1 = _tc2(xyzp[i], gx, g1, wd, _P_KM[i])
            g2 = _sc_gather(fagg1, nt_km[i], seg=N, n_tab=N).reshape(K_N, BN, d2)
            fe = _tc3(xyzp[i], gx, g2, feat, wd, feat.shape[1], _P_KM[i])
        if i == 0:
            fe0 = fe
        N1 = NS[i + 1]
        dprev = 2 * D_OUT[i]
        pg = _sc_gather(fe, st_pm[i], seg=N1 * K_N, n_tab=N
                        ).reshape(B * N1, K_N * dprev)
        if i < NUM_LAYERS - 1:
            feat, fpc = _tc1(pg, p, 'enc%d_' % (i + 1),
                             dprev, D_OUT[i + 1] // 2, _P_TC1[i + 1])
            skips.append(feat)

    f = _tc_dec0(pg, p)
    dec_P = [512, 512, 1024]
    for j in range(NUM_LAYERS - 1):
        lev = NUM_LAYERS - 1 - j
        fi = _sc_gather(f, it[lev], seg=NS[lev], n_tab=NS[lev + 1])
        skip = skips[2 - j]
        nm = 'dec%d' % j
        ds = skip.shape[1]
        f = _tc_dec(skip, fi, params[nm + '_W'][:ds], params[nm + '_W'][ds:],
                    p[nm + '_b2'], p[nm + '_gamma2'], p[nm + '_beta2'], dec_P[j])

    fi = _sc_gather(f, it[0], seg=NS[0], n_tab=NS[1])
    out = _tc_head(p, fe0, fi)
    return out.reshape(B, N0, NUM_CLASSES)
